# baseline (speedup 1.0000x reference)
import functools

import jax
import jax.numpy as jnp
from jax import lax
from jax.experimental import pallas as pl
from jax.experimental.pallas import tpu as pltpu
from jax.experimental.pallas import tpu_sc as plsc

EMBED_DIM = 128
CHUNK = 128
NBUF = 4


@functools.lru_cache(maxsize=None)
def _make_lookup(n_idx: int, vocab: int, dim: int):
    info = plsc.get_sparse_core_info()
    nc, ns = info.num_cores, info.num_subcores
    nw = nc * ns
    assert n_idx % (nw * CHUNK) == 0
    n_chunks = n_idx // (nw * CHUNK)
    assert n_chunks % NBUF == 0 and n_chunks >= 2 * NBUF
    mesh = plsc.VectorSubcoreMesh(core_axis_name="c", subcore_axis_name="s")

    @functools.partial(
        pl.kernel,
        mesh=mesh,
        out_type=jax.ShapeDtypeStruct((nw, n_chunks, CHUNK, dim), jnp.float32),
        scratch_types=[
            pltpu.VMEM((n_chunks, CHUNK), jnp.int32),
            pltpu.VMEM((NBUF, CHUNK, dim), jnp.float32),
            pltpu.SemaphoreType.DMA((NBUF,)),
            pltpu.SemaphoreType.DMA((NBUF,)),
        ],
        compiler_params=pltpu.CompilerParams(use_tc_tiling_on_sc=True),
    )
    def lookup(tok_hbm, table_hbm, out_hbm, idx_v, rows_v, sem_in, sem_out):
        wid = lax.axis_index("s") * nc + lax.axis_index("c")
        pltpu.sync_copy(tok_hbm.at[wid], idx_v)

        def fire_gather(g, p):
            pltpu.make_async_copy(
                table_hbm.at[idx_v.at[g]], rows_v.at[p], sem_in.at[p]
            ).start()

        def wait_gather(p):
            pltpu.make_async_copy(
                table_hbm.at[pl.ds(0, CHUNK)], rows_v.at[p], sem_in.at[p]
            ).wait()

        def fire_scatter(g, p):
            pltpu.make_async_copy(
                rows_v.at[p], out_hbm.at[wid, g], sem_out.at[p]
            ).start()

        def wait_scatter(p):
            pltpu.make_async_copy(
                rows_v.at[p], out_hbm.at[wid, 0], sem_out.at[p]
            ).wait()

        fire_gather(0, 0)
        for g in range(1, NBUF):
            fire_gather(g, g)
            wait_gather(g - 1)
            fire_scatter(g - 1, g - 1)

        def body(i, carry):
            g0 = i * NBUF
            for p in range(NBUF):
                g = g0 + p
                wait_scatter(p)
                fire_gather(g, p)
                pm1 = (p - 1) % NBUF
                wait_gather(pm1)
                fire_scatter(g - 1, pm1)
            return carry

        lax.fori_loop(1, n_chunks // NBUF, body, 0)

        wait_gather(NBUF - 1)
        fire_scatter(n_chunks - 1, NBUF - 1)
        for p in range(NBUF):
            wait_scatter(p)

    return lookup, nw, n_chunks


def kernel(tokens, table):
    b, t = tokens.shape
    vocab, dim = table.shape
    n_idx = b * t
    lookup, nw, n_chunks = _make_lookup(n_idx, vocab, dim)
    tok = tokens.reshape(nw, n_chunks, CHUNK).astype(jnp.int32)
    out = lookup(tok, table)
    return out.reshape(b, t, dim)


# speedup vs baseline: 1.6428x; 1.6428x over previous
import functools

import jax
import jax.numpy as jnp
from jax import lax
from jax.experimental import pallas as pl
from jax.experimental.pallas import tpu as pltpu
from jax.experimental.pallas import tpu_sc as plsc

NBUF = 4


@functools.lru_cache(maxsize=None)
def _make_lookup(n_sent: int, sent_len: int, vocab: int, dim: int):
    info = plsc.get_sparse_core_info()
    nc, ns = info.num_cores, info.num_subcores
    nw = nc * ns
    pad_len = (sent_len + 7) // 8 * 8
    assert n_sent % nw == 0
    s_per_w = n_sent // nw
    assert s_per_w % NBUF == 0 and s_per_w >= 2 * NBUF
    mesh = plsc.VectorSubcoreMesh(core_axis_name="c", subcore_axis_name="s")

    @functools.partial(
        pl.kernel,
        mesh=mesh,
        out_type=jax.ShapeDtypeStruct((n_sent, sent_len, dim), jnp.float32),
        scratch_types=[
            pltpu.VMEM((s_per_w * pad_len,), jnp.int32),
            pltpu.VMEM((NBUF, pad_len, dim), jnp.float32),
            pltpu.SemaphoreType.DMA((NBUF,)),
            pltpu.SemaphoreType.DMA((NBUF,)),
        ],
        compiler_params=pltpu.CompilerParams(use_tc_tiling_on_sc=True),
    )
    def lookup(tok_hbm, table_hbm, out_hbm, idx_v, rows_v, sem_in, sem_out):
        wid = lax.axis_index("s") * nc + lax.axis_index("c")
        base = wid * s_per_w
        pltpu.sync_copy(tok_hbm.at[pl.ds(base * pad_len, s_per_w * pad_len)], idx_v)

        def fire_gather(s, p):
            pltpu.make_async_copy(
                table_hbm.at[idx_v.at[pl.ds(s * pad_len, pad_len)]],
                rows_v.at[p],
                sem_in.at[p],
            ).start()

        def wait_gather(p):
            pltpu.make_async_copy(
                table_hbm.at[pl.ds(0, pad_len)], rows_v.at[p], sem_in.at[p]
            ).wait()

        def fire_scatter(s, p):
            pltpu.make_async_copy(
                rows_v.at[p, pl.ds(0, sent_len)],
                out_hbm.at[base + s],
                sem_out.at[p],
            ).start()

        def wait_scatter(p):
            pltpu.make_async_copy(
                rows_v.at[p, pl.ds(0, sent_len)], out_hbm.at[0], sem_out.at[p]
            ).wait()

        fire_gather(0, 0)
        for s in range(1, NBUF):
            fire_gather(s, s)
            wait_gather(s - 1)
            fire_scatter(s - 1, s - 1)

        def body(i, carry):
            s0 = i * NBUF
            for p in range(NBUF):
                s = s0 + p
                wait_scatter(p)
                fire_gather(s, p)
                pm1 = (p - 1) % NBUF
                wait_gather(pm1)
                fire_scatter(s - 1, pm1)
            return carry

        lax.fori_loop(1, s_per_w // NBUF, body, 0)

        wait_gather(NBUF - 1)
        fire_scatter(s_per_w - 1, NBUF - 1)
        for p in range(NBUF):
            wait_scatter(p)

    return lookup, pad_len


def kernel(tokens, table):
    n_sent, sent_len = tokens.shape
    vocab, dim = table.shape
    lookup, pad_len = _make_lookup(n_sent, sent_len, vocab, dim)
    tok = tokens.astype(jnp.int32)
    tok = jnp.pad(tok, ((0, 0), (0, pad_len - sent_len)), mode="edge")
    return lookup(tok.reshape(-1), table)


# speedup vs baseline: 3.0786x; 1.8740x over previous
import functools

import jax
import jax.numpy as jnp
from jax import lax
from jax.experimental import pallas as pl
from jax.experimental.pallas import tpu as pltpu
from jax.experimental.pallas import tpu_sc as plsc

CHUNK = 128
NBUF = 6
LOOK = 3


@functools.lru_cache(maxsize=None)
def _make_lookup(n_idx: int, vocab: int, dim: int):
    info = plsc.get_sparse_core_info()
    nc, ns = info.num_cores, info.num_subcores
    nw = nc * ns
    assert n_idx % (nw * CHUNK) == 0
    n_chunks = n_idx // (nw * CHUNK)
    n_steady = (n_chunks - NBUF - LOOK - 1) // NBUF * NBUF
    assert n_steady > 0
    mesh = plsc.VectorSubcoreMesh(core_axis_name="c", subcore_axis_name="s")

    @functools.partial(
        pl.kernel,
        mesh=mesh,
        out_type=jax.ShapeDtypeStruct((n_idx, dim), jnp.float32),
        scratch_types=[
            pltpu.VMEM((n_chunks, CHUNK), jnp.int32),
            pltpu.VMEM((NBUF, CHUNK, dim), jnp.float32),
            pltpu.SemaphoreType.DMA((NBUF,)),
            pltpu.SemaphoreType.DMA((NBUF,)),
        ],
        compiler_params=pltpu.CompilerParams(use_tc_tiling_on_sc=True),
    )
    def lookup(tok_hbm, table_hbm, out_hbm, idx_v, rows_v, sem_in, sem_out):
        wid = lax.axis_index("s") * nc + lax.axis_index("c")
        base = wid * (n_chunks * CHUNK)
        pltpu.sync_copy(tok_hbm.at[wid], idx_v)

        def fire_gather(g, p):
            pltpu.make_async_copy(
                table_hbm.at[idx_v.at[g]], rows_v.at[p], sem_in.at[p]
            ).start()

        def wait_gather(p):
            pltpu.make_async_copy(
                table_hbm.at[pl.ds(0, CHUNK)], rows_v.at[p], sem_in.at[p]
            ).wait()

        def fire_scatter(g, p):
            pltpu.make_async_copy(
                rows_v.at[p], out_hbm.at[pl.ds(base + g * CHUNK, CHUNK)], sem_out.at[p]
            ).start()

        def wait_scatter(p):
            pltpu.make_async_copy(
                rows_v.at[p], out_hbm.at[pl.ds(0, CHUNK)], sem_out.at[p]
            ).wait()

        def step(g, p, q, refill, drain):
            wait_gather(p)
            fire_scatter(g, p)
            if drain:
                wait_scatter(q)
            if refill:
                fire_gather(g + LOOK, q)

        for g in range(LOOK):
            fire_gather(g, g)
        head = n_chunks - LOOK - n_steady - (NBUF - LOOK)
        for g in range(head):
            step(g, g % NBUF, (g + LOOK) % NBUF, True, g + LOOK >= NBUF)

        def body(i, carry):
            g0 = head + i * NBUF
            for r in range(NBUF):
                g = g0 + r
                p = (head + r) % NBUF
                q = (head + r + LOOK) % NBUF
                step(g, p, q, True, True)
            return carry

        lax.fori_loop(0, n_steady // NBUF, body, 0)

        for g in range(head + n_steady, n_chunks - LOOK):
            step(g, g % NBUF, (g + LOOK) % NBUF, True, True)
        for g in range(n_chunks - LOOK, n_chunks):
            wait_gather(g % NBUF)
            fire_scatter(g, g % NBUF)
        for g in range(n_chunks - NBUF, n_chunks):
            wait_scatter(g % NBUF)

    return lookup, nw, n_chunks


def kernel(tokens, table):
    b, t = tokens.shape
    vocab, dim = table.shape
    n_idx = b * t
    lookup, nw, n_chunks = _make_lookup(n_idx, vocab, dim)
    tok = tokens.T.astype(jnp.int32).reshape(nw, n_chunks, CHUNK)
    out = lookup(tok, table)
    return out.reshape(t, b, dim).transpose(1, 0, 2)
